# Initial kernel scaffold; baseline (speedup 1.0000x reference)
import jax
import jax.numpy as jnp
from jax.experimental import pallas as pl


def kernel(points, featmap_xy, featmap_yz, featmap_xz):
    raise NotImplementedError("write your pallas kernel here")


# speedup vs baseline: 1.8727x; 1.8727x over previous
import functools

import jax
import jax.numpy as jnp
from jax import lax
from jax.experimental import pallas as pl
from jax.experimental.pallas import tpu as pltpu
from jax.experimental.pallas import tpu_sc as plsc

_RES = 512
_NCH = 32
_N = 4 * 1024 * 64
_CHUNK = 128
_NW = 32
_CHUNKS_PER_W = _N // _CHUNK // _NW


def _prep_body(fxy, fyz, fxz, oxy, oyz, oxz):
    for s, d in ((fxy, oxy), (fyz, oyz), (fxz, oxz)):
        t = jnp.tanh(s[0])
        t = t.reshape(_NCH, -1)
        d[...] = t.T


def _prep_tables(fxy, fyz, fxz):
    rows = 8
    grid = _RES // rows
    spec_in = pl.BlockSpec((1, _NCH, rows, _RES), lambda i: (0, 0, i, 0))
    spec_out = pl.BlockSpec((rows * _RES, _NCH), lambda i: (i, 0))
    out_sd = jax.ShapeDtypeStruct((_RES * _RES, _NCH), jnp.float32)
    return pl.pallas_call(
        _prep_body,
        grid=(grid,),
        in_specs=[spec_in] * 3,
        out_specs=[spec_out] * 3,
        out_shape=[out_sd] * 3,
    )(fxy, fyz, fxz)


def _axis_prep(v):
    i = jnp.clip((v + 1.0) * 0.5 * (_RES - 1.0), 0.0, _RES - 1.0)
    i0 = i.astype(jnp.int32)
    f = i - i0.astype(jnp.float32)
    i1 = jnp.minimum(i0 + 1, _RES - 1)
    return i0, i1, f


def _sc_body(xs, ys, zs, txy, tyz, txz, out, xv, yv, zv, idxv, wv, rowsv,
             obuf, sem):
    wid = lax.axis_index("s") * 2 + lax.axis_index("c")

    def chunk_body(k, carry):
        base = (wid * _CHUNKS_PER_W + k) * _CHUNK
        pltpu.sync_copy(xs.at[pl.ds(base, _CHUNK)], xv)
        pltpu.sync_copy(ys.at[pl.ds(base, _CHUNK)], yv)
        pltpu.sync_copy(zs.at[pl.ds(base, _CHUNK)], zv)
        for g in range(_CHUNK // 16):
            sl = pl.ds(g * 16, 16)
            ax = _axis_prep(xv[sl])
            ay = _axis_prep(yv[sl])
            az = _axis_prep(zv[sl])
            for p, (col, row) in enumerate(((ax, ay), (ay, az), (ax, az))):
                c0, c1, fc = col
                r0, r1, fr = row
                rb0 = r0 * _RES
                rb1 = r1 * _RES
                gc = 1.0 - fc
                gr = 1.0 - fr
                j = 4 * p
                idxv[j + 0, sl] = rb0 + c0
                idxv[j + 1, sl] = rb0 + c1
                idxv[j + 2, sl] = rb1 + c0
                idxv[j + 3, sl] = rb1 + c1
                wv[j + 0, sl] = gr * gc
                wv[j + 1, sl] = gr * fc
                wv[j + 2, sl] = fr * gc
                wv[j + 3, sl] = fr * fc
        cps = []
        for p, tbl in enumerate((txy, tyz, txz)):
            for t in range(4):
                j = 4 * p + t
                cps.append(pltpu.async_copy(tbl.at[idxv.at[j]], rowsv.at[j],
                                            sem))
        for cp in cps:
            cp.wait()

        def pt_body(g, c2):
            gb = g * 16
            wlane = [wv[j, pl.ds(gb, 16)] for j in range(12)]
            for l in range(16):
                i = gb + l
                for p in range(3):
                    a0 = None
                    a1 = None
                    for t in range(4):
                        j = 4 * p + t
                        w = wlane[j][l]
                        v0 = w * rowsv[j, i, pl.ds(0, 16)]
                        v1 = w * rowsv[j, i, pl.ds(16, 16)]
                        a0 = v0 if a0 is None else a0 + v0
                        a1 = v1 if a1 is None else a1 + v1
                    obuf[i, pl.ds(32 * p, 16)] = a0
                    obuf[i, pl.ds(32 * p + 16, 16)] = a1
            return c2

        lax.fori_loop(0, _CHUNK // 16, pt_body, None)
        pltpu.sync_copy(obuf, out.at[pl.ds(base, _CHUNK)])
        return carry

    lax.fori_loop(0, _CHUNKS_PER_W, chunk_body, None)


_sc_gather = functools.partial(
    pl.kernel,
    out_type=jax.ShapeDtypeStruct((_N, 96), jnp.float32),
    mesh=plsc.VectorSubcoreMesh(core_axis_name="c", subcore_axis_name="s"),
    compiler_params=pltpu.CompilerParams(use_tc_tiling_on_sc=False),
    scratch_types=[
        pltpu.VMEM((_CHUNK,), jnp.float32),
        pltpu.VMEM((_CHUNK,), jnp.float32),
        pltpu.VMEM((_CHUNK,), jnp.float32),
        pltpu.VMEM((12, _CHUNK), jnp.int32),
        pltpu.VMEM((12, _CHUNK), jnp.float32),
        pltpu.VMEM((12, _CHUNK, _NCH), jnp.float32),
        pltpu.VMEM((_CHUNK, 96), jnp.float32),
        pltpu.SemaphoreType.DMA,
    ],
)(_sc_body)


def kernel(points, featmap_xy, featmap_yz, featmap_xz):
    txy, tyz, txz = _prep_tables(featmap_xy, featmap_yz, featmap_xz)
    pts = points.reshape(_N, 3)
    xs = pts[:, 0]
    ys = pts[:, 1]
    zs = pts[:, 2]
    out = _sc_gather(xs, ys, zs, txy, tyz, txz)
    out = out.reshape(4, 1024, 64, 96)
    return jnp.transpose(out, (0, 3, 1, 2))
